# baseline (speedup 1.0000x reference)
import functools

import jax
import jax.numpy as jnp
from jax import lax
from jax.experimental import pallas as pl
from jax.experimental.pallas import tpu as pltpu
from jax.experimental.pallas import tpu_sc as plsc

_N = 10000
_E = 320000
_H = 128
_NORM_INV = 0.01
_EPS_LN = 1e-5
_EPS_R = 1e-8

_NC = 2
_NS = 16
_NW = _NC * _NS
_L = 16
_C = 128
_NCH = _E // _C
_BASE_CH = _NCH // _NW
_EXTRA = _NCH - _BASE_CH * _NW
_RPT = 624
_CW = 8


def _mesh():
    return plsc.VectorSubcoreMesh(core_axis_name="c", subcore_axis_name="s")


def _worker_id():
    return lax.axis_index("s") * _NC + lax.axis_index("c")


def _foreach_chunk(do_chunk):
    w = _worker_id()

    def body(k, carry):
        do_chunk(w + k * _NW)
        return carry

    lax.fori_loop(0, _BASE_CH, body, 0)

    @pl.when(w < _EXTRA)
    def _():
        do_chunk(_BASE_CH * _NW + w)


def _geom_body(ct_hbm, row_hbm, col_hbm, geom_o, idx_r, idx_c, ct_v, bg, sem):
    pltpu.sync_copy(ct_hbm, ct_v)
    lanes = jnp.arange(_L, dtype=jnp.int32)

    def do_chunk(ci):
        base = ci * _C
        i1 = pltpu.async_copy(row_hbm.at[pl.ds(base, _C)], idx_r, sem)
        i2 = pltpu.async_copy(col_hbm.at[pl.ds(base, _C)], idx_c, sem)
        i1.wait()
        i2.wait()
        for j in range(_C // _L):
            r16 = idx_r[pl.ds(j * _L, _L)] * _CW
            c16 = idx_c[pl.ds(j * _L, _L)] * _CW
            a0 = plsc.load_gather(ct_v, [r16])
            a1 = plsc.load_gather(ct_v, [r16 + 1])
            a2 = plsc.load_gather(ct_v, [r16 + 2])
            b0 = plsc.load_gather(ct_v, [c16])
            b1 = plsc.load_gather(ct_v, [c16 + 1])
            b2 = plsc.load_gather(ct_v, [c16 + 2])
            d0 = a0 - b0
            d1 = a1 - b1
            d2 = a2 - b2
            rad = d0 * d0 + d1 * d1 + d2 * d2
            eb = (j * _L + lanes) * _CW
            plsc.store_scatter(bg, [eb], d0)
            plsc.store_scatter(bg, [eb + 1], d1)
            plsc.store_scatter(bg, [eb + 2], d2)
            plsc.store_scatter(bg, [eb + 3], rad)
        pltpu.sync_copy(bg, geom_o.at[pl.ds(base * _CW, _C * _CW)])

    _foreach_chunk(do_chunk)


def _sc_geom(ctab, row, col):
    f = functools.partial(
        pl.kernel, mesh=_mesh(),
        out_type=jax.ShapeDtypeStruct((_E * _CW,), jnp.float32),
        compiler_params=pltpu.CompilerParams(needs_layout_passes=False),
        scratch_types=[
            pltpu.VMEM((_C,), jnp.int32),
            pltpu.VMEM((_C,), jnp.int32),
            pltpu.VMEM((_N * _CW,), jnp.float32),
            pltpu.VMEM((_C * _CW,), jnp.float32),
            pltpu.SemaphoreType.DMA,
        ],
    )(_geom_body)
    return f(ctab, row, col)


_NB = 3
_GRP = _BASE_CH // _NB

_EH = _E // 2
_HCH = _EH // _C
_HBASE = _HCH // _NW
_HGRP = _HBASE // _NB
_HEXTRA = _HCH - _HBASE * _NW


def _g2_body(h_hbm, row_hbm, col_hbm, src_o, tgt_o,
             idx_r, idx_c, bs, bt, isem, gsem, ws0, ws1, ws2):
    grp, nbase, extra = _GRP, _BASE_CH, _EXTRA
    w = _worker_id()
    wsems = (ws0, ws1, ws2)

    def drain(j):
        pltpu.make_async_copy(bs.at[j], src_o.at[pl.ds(0, _C)], wsems[j]).wait()
        pltpu.make_async_copy(bt.at[j], tgt_o.at[pl.ds(0, _C)], wsems[j]).wait()

    def body(m, carry):
        ih = []
        for j in range(_NB):
            ci = w + (_NB * m + j) * _NW
            base = ci * _C

            @pl.when(m > 0)
            def _():
                drain(j)

            i1 = pltpu.async_copy(row_hbm.at[pl.ds(base, _C)], idx_r.at[j], isem)
            i2 = pltpu.async_copy(col_hbm.at[pl.ds(base, _C)], idx_c.at[j], isem)
            ih.append((i1, i2, base))
        handles = []
        for j in range(_NB):
            i1, i2, base = ih[j]
            i1.wait()
            i2.wait()
            g1 = pltpu.async_copy(h_hbm.at[idx_r.at[j]], bs.at[j], gsem)
            g2 = pltpu.async_copy(h_hbm.at[idx_c.at[j]], bt.at[j], gsem)
            handles.append((g1, g2, base))
        for j in range(_NB):
            g1, g2, base = handles[j]
            g1.wait()
            g2.wait()
            pltpu.async_copy(bs.at[j], src_o.at[pl.ds(base, _C)], wsems[j])
            pltpu.async_copy(bt.at[j], tgt_o.at[pl.ds(base, _C)], wsems[j])
        return carry

    lax.fori_loop(0, grp, body, 0)
    for j in range(_NB):
        drain(j)

    @pl.when(w < extra)
    def _():
        base = (nbase * _NW + w) * _C
        pltpu.sync_copy(row_hbm.at[pl.ds(base, _C)], idx_r.at[0])
        pltpu.sync_copy(col_hbm.at[pl.ds(base, _C)], idx_c.at[0])
        g1 = pltpu.async_copy(h_hbm.at[idx_r.at[0]], bs.at[0], gsem)
        g2 = pltpu.async_copy(h_hbm.at[idx_c.at[0]], bt.at[0], gsem)
        g1.wait()
        g2.wait()
        pltpu.sync_copy(bs.at[0], src_o.at[pl.ds(base, _C)])
        pltpu.sync_copy(bt.at[0], tgt_o.at[pl.ds(base, _C)])


def _gather2(h, row, col):
    f = functools.partial(
        pl.kernel, mesh=_mesh(),
        out_type=[
            jax.ShapeDtypeStruct((_E, _H), jnp.float32),
            jax.ShapeDtypeStruct((_E, _H), jnp.float32),
        ],
        scratch_types=[
            pltpu.VMEM((_NB, _C), jnp.int32),
            pltpu.VMEM((_NB, _C), jnp.int32),
            pltpu.VMEM((_NB, _C, _H), jnp.float32),
            pltpu.VMEM((_NB, _C, _H), jnp.float32),
            pltpu.SemaphoreType.DMA,
            pltpu.SemaphoreType.DMA,
            pltpu.SemaphoreType.DMA,
            pltpu.SemaphoreType.DMA,
            pltpu.SemaphoreType.DMA,
        ],
    )(_g2_body)
    return f(h, row, col)


def _scatter_body(ef_hbm, row_hbm, z_hbm, out_hbm, idx_v, buf, acc,
                  isem, lsem, ss0, ss1, ss2):
    c = lax.axis_index("c")
    s = lax.axis_index("s")
    start = s * _RPT

    @pl.when(s < _NS - 1)
    def _():
        pltpu.sync_copy(z_hbm.at[pl.ds(start, _RPT)],
                        acc.at[pl.ds(start, _RPT)])

    @pl.when(s == _NS - 1)
    def _():
        pltpu.sync_copy(z_hbm.at[pl.ds((_NS - 1) * _RPT, _N - (_NS - 1) * _RPT)],
                        acc.at[pl.ds((_NS - 1) * _RPT, _N - (_NS - 1) * _RPT)])

    plsc.subcore_barrier()

    w = s * _NC + c
    ssems = (ss0, ss1, ss2)

    def drain(j):
        pltpu.make_async_copy(buf.at[j], acc.at[pl.ds(0, _C)], ssems[j]).wait()

    def body(m, carry):
        handles = []
        for j in range(_NB):
            ci = w + (_NB * m + j) * _NW
            base = ci * _C

            @pl.when(m > 0)
            def _():
                drain(j)

            i1 = pltpu.async_copy(row_hbm.at[pl.ds(base, _C)], idx_v.at[j], isem)
            h1 = pltpu.async_copy(ef_hbm.at[pl.ds(base, _C)], buf.at[j], lsem)
            handles.append((i1, h1))
        for j in range(_NB):
            i1, h1 = handles[j]
            i1.wait()
            h1.wait()
            pltpu.async_copy(buf.at[j], acc.at[idx_v.at[j]], ssems[j], add=True)
        return carry

    lax.fori_loop(0, _GRP, body, 0)
    for j in range(_NB):
        drain(j)

    @pl.when(w < _EXTRA)
    def _():
        base = (_BASE_CH * _NW + w) * _C
        pltpu.sync_copy(row_hbm.at[pl.ds(base, _C)], idx_v.at[0])
        pltpu.sync_copy(ef_hbm.at[pl.ds(base, _C)], buf.at[0])
        pltpu.sync_copy(buf.at[0], acc.at[idx_v.at[0]], add=True)

    plsc.subcore_barrier()

    @pl.when(s < _NS - 1)
    def _():
        pltpu.sync_copy(acc.at[pl.ds(start, _RPT)],
                        out_hbm.at[c, pl.ds(start, _RPT)])

    @pl.when(s == _NS - 1)
    def _():
        pltpu.sync_copy(acc.at[pl.ds((_NS - 1) * _RPT, _N - (_NS - 1) * _RPT)],
                        out_hbm.at[c, pl.ds((_NS - 1) * _RPT, _N - (_NS - 1) * _RPT)])


def _scatter_add(ef, row, zeros):
    f = functools.partial(
        pl.kernel, mesh=_mesh(),
        out_type=jax.ShapeDtypeStruct((_NC, _N, _H), jnp.float32),
        scratch_types=[
            pltpu.VMEM((_NB, _C), jnp.int32),
            pltpu.VMEM((_NB, _C, _H), jnp.float32),
            pltpu.VMEM_SHARED((_N, _H), jnp.float32),
            pltpu.SemaphoreType.DMA,
            pltpu.SemaphoreType.DMA,
            pltpu.SemaphoreType.DMA,
            pltpu.SemaphoreType.DMA,
            pltpu.SemaphoreType.DMA,
        ],
    )(_scatter_body)
    return f(ef, row, zeros)


_NR = 640
_RPTC = _NR // _NS


def _cscat_body(t_hbm, g_hbm, row_hbm, z_hbm, out_hbm,
                rb0, rb1, rb2, ix0, ix1, ix2, tb0, tb1, tb2,
                gb0, gb1, gb2, sb0, sb1, sb2, acc, isem, lsem, ss0, ss1, ss2):
    c = lax.axis_index("c")
    s = lax.axis_index("s")
    w = s * _NC + c
    lanes = jnp.arange(_L, dtype=jnp.int32)
    zeros16 = jnp.zeros((_L,), jnp.float32)
    rbs = (rb0, rb1, rb2)
    ixs = (ix0, ix1, ix2)
    tbs = (tb0, tb1, tb2)
    gbs = (gb0, gb1, gb2)
    sbs = (sb0, sb1, sb2)
    ssems = (ss0, ss1, ss2)

    pltpu.sync_copy(z_hbm.at[pl.ds(s * _RPTC, _RPTC)],
                    acc.at[pl.ds(s * _RPTC, _RPTC)])
    for j in range(_NB):
        pltpu.sync_copy(z_hbm.at[pl.ds(0, _C)], sbs[j])
    plsc.subcore_barrier()

    def drain(j):
        pltpu.make_async_copy(sbs[j], acc.at[pl.ds(0, _C)], ssems[j]).wait()

    def unfill(j):
        for g in range(_C // _L):
            e16 = g * _L + lanes
            r16 = rbs[j][pl.ds(g * _L, _L)]
            laneb = (r16 & 15) * _CW
            for cc in range(3):
                plsc.store_scatter(sbs[j], [e16, laneb + cc], zeros16)

    def fill(j):
        for g in range(_C // _L):
            e16 = g * _L + lanes
            r16 = rbs[j][pl.ds(g * _L, _L)]
            ixs[j][pl.ds(g * _L, _L)] = lax.shift_right_logical(r16, 4)
            laneb = (r16 & 15) * _CW
            t16 = tbs[j][pl.ds(g * _L, _L)]
            for cc in range(3):
                dcc = plsc.load_gather(gbs[j], [e16 * _CW + cc])
                plsc.store_scatter(sbs[j], [e16, laneb + cc], dcc * t16)

    def body(m, carry):
        handles = []
        for j in range(_NB):
            ci = w + (_NB * m + j) * _NW
            base = ci * _C

            @pl.when(m > 0)
            def _():
                drain(j)
                unfill(j)

            i1 = pltpu.async_copy(row_hbm.at[pl.ds(base, _C)], rbs[j], isem)
            h1 = pltpu.async_copy(t_hbm.at[pl.ds(base, _C)], tbs[j], lsem)
            h2 = pltpu.async_copy(g_hbm.at[pl.ds(base * _CW, _C * _CW)],
                                  gbs[j], lsem)
            handles.append((i1, h1, h2))
        for j in range(_NB):
            i1, h1, h2 = handles[j]
            i1.wait()
            h1.wait()
            h2.wait()
            fill(j)
            pltpu.async_copy(sbs[j], acc.at[ixs[j]], ssems[j], add=True)
        return carry

    lax.fori_loop(0, _GRP, body, 0)
    for j in range(_NB):
        drain(j)

    @pl.when(w < _EXTRA)
    def _():
        base = (_BASE_CH * _NW + w) * _C
        unfill(0)
        pltpu.sync_copy(row_hbm.at[pl.ds(base, _C)], rbs[0])
        pltpu.sync_copy(t_hbm.at[pl.ds(base, _C)], tbs[0])
        pltpu.sync_copy(g_hbm.at[pl.ds(base * _CW, _C * _CW)], gbs[0])
        fill(0)
        pltpu.sync_copy(sbs[0], acc.at[ixs[0]], add=True)

    plsc.subcore_barrier()
    pltpu.sync_copy(acc.at[pl.ds(s * _RPTC, _RPTC)],
                    out_hbm.at[c, pl.ds(s * _RPTC, _RPTC)])


def _coord_scatter(tprime, geom1d, row, zeros):
    f = functools.partial(
        pl.kernel, mesh=_mesh(),
        out_type=jax.ShapeDtypeStruct((_NC, _NR, _H), jnp.float32),
        compiler_params=pltpu.CompilerParams(needs_layout_passes=False),
        scratch_types=(
            [pltpu.VMEM((_C,), jnp.int32)] * _NB
            + [pltpu.VMEM((_C,), jnp.int32)] * _NB
            + [pltpu.VMEM((_C,), jnp.float32)] * _NB
            + [pltpu.VMEM((_C * _CW,), jnp.float32)] * _NB
            + [pltpu.VMEM((_C, _H), jnp.float32)] * _NB
            + [pltpu.VMEM_SHARED((_NR, _H), jnp.float32)]
            + [pltpu.SemaphoreType.DMA] * 5
        ),
    )(_cscat_body)
    return f(tprime, geom1d, row, zeros)


_BE = 1280
_BN = 1000


def _ln_stats(x):
    o = jnp.ones((_H, 1), jnp.float32)
    m = jnp.dot(x, o, preferred_element_type=jnp.float32) * (1.0 / _H)
    s2 = jnp.dot(x * x, o, preferred_element_type=jnp.float32) * (1.0 / _H)
    return m, s2 - m * m


def _edge_body(src_ref, tgt_ref, geom_ref, ea_ref,
               A_ref, B_ref, b1_ref, ar_ref, ae_ref, g1_ref, bg1_ref,
               W2_ref, b2_ref, aw_ref, ab_ref, out_ref):
    radial = geom_ref[...][:, 3:4]
    x = jnp.dot(src_ref[...], A_ref[...], preferred_element_type=jnp.float32)
    x = x + jnp.dot(tgt_ref[...], B_ref[...], preferred_element_type=jnp.float32)
    x = x + radial * ar_ref[...] + ea_ref[...] * ae_ref[...] + b1_ref[...]
    m, v = _ln_stats(x)
    x = (x - m) * lax.rsqrt(v + _EPS_LN) * g1_ref[...] + bg1_ref[...]
    x = x * jax.nn.sigmoid(x)
    y = jnp.dot(x, W2_ref[...],
                preferred_element_type=jnp.float32) + b2_ref[...]
    y = y * jax.nn.sigmoid(y)
    att = jax.nn.sigmoid(
        jnp.dot(y, aw_ref[...], preferred_element_type=jnp.float32)
        + ab_ref[...])
    out_ref[...] = y * att


def _tc_edge(src, tgt, geom, ea, A, B, b1, ar, ae, g1, bg1, W2, b2, aw, ab):
    im = lambda i: (i, 0)
    full = lambda shape: pl.BlockSpec(shape, lambda i: (0, 0))
    return pl.pallas_call(
        _edge_body,
        grid=(_E // _BE,),
        in_specs=[
            pl.BlockSpec((_BE, _H), im), pl.BlockSpec((_BE, _H), im),
            pl.BlockSpec((_BE, _CW), im), pl.BlockSpec((_BE, 1), im),
            full((_H, _H)), full((_H, _H)), full((1, _H)), full((1, _H)),
            full((1, _H)), full((1, _H)), full((1, _H)),
            full((_H, _H)), full((1, _H)), full((_H, 1)), full((1, 1)),
        ],
        out_specs=pl.BlockSpec((_BE, _H), im),
        out_shape=jax.ShapeDtypeStruct((_E, _H), jnp.float32),
    )(src, tgt, geom, ea, A, B, b1, ar, ae, g1, bg1, W2, b2, aw, ab)


def _node_body(h_ref, p0_ref, p1_ref, p2_ref, p3_ref,
               Wh_ref, Wa_ref, b1_ref, g_ref, bg_ref, W2_ref, b2_ref, out_ref):
    h = h_ref[...]
    agg = ((p0_ref[...] + p1_ref[...]) + (p2_ref[...] + p3_ref[...])) * _NORM_INV
    x = jnp.dot(h, Wh_ref[...], preferred_element_type=jnp.float32)
    x = x + jnp.dot(agg, Wa_ref[...], preferred_element_type=jnp.float32) + b1_ref[...]
    m = jnp.mean(x, axis=-1, keepdims=True)
    v = jnp.mean((x - m) ** 2, axis=-1, keepdims=True)
    x = (x - m) * lax.rsqrt(v + _EPS_LN) * g_ref[...] + bg_ref[...]
    x = x * jax.nn.sigmoid(x)
    nu = jnp.dot(x, W2_ref[...], preferred_element_type=jnp.float32) + b2_ref[...]
    out_ref[...] = h + nu


def _tc_node(h, pa, pb, Wh, Wa, b1, g, bg, W2, b2):
    im = lambda i: (i, 0)
    full = lambda shape: pl.BlockSpec(shape, lambda i: (0, 0))
    return pl.pallas_call(
        _node_body,
        grid=(_N // _BN,),
        in_specs=[
            pl.BlockSpec((_BN, _H), im), pl.BlockSpec((_BN, _H), im),
            pl.BlockSpec((_BN, _H), im), pl.BlockSpec((_BN, _H), im),
            pl.BlockSpec((_BN, _H), im),
            full((_H, _H)), full((_H, _H)), full((1, _H)), full((1, _H)),
            full((1, _H)), full((_H, _H)), full((1, _H)),
        ],
        out_specs=pl.BlockSpec((_BN, _H), im),
        out_shape=jax.ShapeDtypeStruct((_N, _H), jnp.float32),
    )(h, pa[0], pa[1], pb[0], pb[1], Wh, Wa, b1, g, bg, W2, b2)


def _eq_body(src_ref, tgt_ref, geom_ref, ea_ref,
             A_ref, B_ref, b1_ref, ar_ref, ae_ref, g1_ref, bg1_ref,
             W2_ref, b2_ref, g2_ref, bg2_ref, w3_ref, out_ref):
    geom = geom_ref[...]
    radial = geom[:, 3:4]
    x = jnp.dot(src_ref[...], A_ref[...], preferred_element_type=jnp.float32)
    x = x + jnp.dot(tgt_ref[...], B_ref[...], preferred_element_type=jnp.float32)
    x = x + radial * ar_ref[...] + ea_ref[...] * ae_ref[...] + b1_ref[...]
    m, v = _ln_stats(x)
    x = (x - m) * lax.rsqrt(v + _EPS_LN) * g1_ref[...] + bg1_ref[...]
    x = x * jax.nn.sigmoid(x)
    y = jnp.dot(x, W2_ref[...],
                preferred_element_type=jnp.float32) + b2_ref[...]
    m, v = _ln_stats(y)
    y = (y - m) * lax.rsqrt(v + _EPS_LN) * g2_ref[...] + bg2_ref[...]
    y = y * jax.nn.sigmoid(y)
    t = jnp.dot(y, w3_ref[...], preferred_element_type=jnp.float32)
    out_ref[...] = t / (jnp.sqrt(radial + _EPS_R) + 1.0)


def _tc_eq(src, tgt, geom, ea, A, B, b1, ar, ae, g1, bg1, W2, b2, g2, bg2, w3):
    im = lambda i: (i, 0)
    full = lambda shape: pl.BlockSpec(shape, lambda i: (0, 0))
    return pl.pallas_call(
        _eq_body,
        grid=(_E // _BE,),
        in_specs=[
            pl.BlockSpec((_BE, _H), im), pl.BlockSpec((_BE, _H), im),
            pl.BlockSpec((_BE, _CW), im), pl.BlockSpec((_BE, 1), im),
            full((_H, _H)), full((_H, _H)), full((1, _H)), full((1, _H)),
            full((1, _H)), full((1, _H)), full((1, _H)),
            full((_H, _H)), full((1, _H)), full((1, _H)), full((1, _H)),
            full((_H, 1)),
        ],
        out_specs=pl.BlockSpec((_BE, 1), im),
        out_shape=jax.ShapeDtypeStruct((_E, 1), jnp.float32),
    )(src, tgt, geom, ea, A, B, b1, ar, ae, g1, bg1, W2, b2, g2, bg2, w3)


def _coord_body(cp_ref, p0_ref, p1_ref, p2_ref, p3_ref, out_ref):
    out_ref[...] = cp_ref[...] + (
        (p0_ref[...] + p1_ref[...]) + (p2_ref[...] + p3_ref[...])) * _NORM_INV


def _tc_coord(cpad, p0, p1, p2, p3):
    im = lambda i: (i, 0)
    return pl.pallas_call(
        _coord_body,
        grid=(_N // _BN,),
        in_specs=[pl.BlockSpec((_BN, _CW), im)] * 5,
        out_specs=pl.BlockSpec((_BN, _CW), im),
        out_shape=jax.ShapeDtypeStruct((_N, _CW), jnp.float32),
    )(cpad, p0, p1, p2, p3)


def _edge_params(p, w1_key='e_w1', b1_key='e_b1', g_key='e_ln_g', bg_key='e_ln_b',
                 w2_key='e_w2', b2_key='e_b2'):
    W1 = p[w1_key]
    return dict(
        A=W1[:_H], B=W1[_H:2 * _H],
        ar=W1[2 * _H:2 * _H + 1], ae=W1[2 * _H + 1:2 * _H + 2],
        b1=p[b1_key].reshape(1, _H), g1=p[g_key].reshape(1, _H),
        bg1=p[bg_key].reshape(1, _H),
        W2=p[w2_key], b2=p[b2_key].reshape(1, _H),
    )


def kernel(h, coord, edge_attr, params, edge_index):
    row = edge_index[0]
    col = edge_index[1]
    ctab = jnp.pad(coord, ((0, 0), (0, _CW - 3))).reshape(-1)
    zeros_h = jnp.zeros((_N, _H), jnp.float32)

    geom1d = _sc_geom(ctab, row, col)
    geom = geom1d.reshape(_E, _CW)

    for i in range(2):
        p = params['gcl%d' % i]
        ep = _edge_params(p)
        src, tgt = _gather2(h, row, col)
        ef = _tc_edge(src, tgt, geom, edge_attr,
                      ep['A'], ep['B'], ep['b1'],
                      ep['ar'], ep['ae'], ep['g1'], ep['bg1'],
                      ep['W2'], ep['b2'],
                      p['att_w'], p['att_b'].reshape(1, 1))
        part = _scatter_add(ef, row, zeros_h)
        h = _tc_node(h, part, part,
                     p['n_w1'][:_H], 0.5 * p['n_w1'][_H:], p['n_b1'].reshape(1, _H),
                     p['n_ln_g'].reshape(1, _H), p['n_ln_b'].reshape(1, _H),
                     p['n_w2'], p['n_b2'].reshape(1, _H))

    eq = params['eq']
    eqp = _edge_params(eq, w1_key='w1', b1_key='b1', g_key='ln1_g', bg_key='ln1_b',
                       w2_key='w2', b2_key='b2')
    src, tgt = _gather2(h, row, col)
    tprime = _tc_eq(src, tgt, geom, edge_attr,
                    eqp['A'], eqp['B'], eqp['b1'],
                    eqp['ar'], eqp['ae'], eqp['g1'], eqp['bg1'],
                    eqp['W2'], eqp['b2'],
                    eq['ln2_g'].reshape(1, _H), eq['ln2_b'].reshape(1, _H),
                    eq['w3'])
    pc = (_coord_scatter(tprime.reshape(_E), geom1d, row, zeros_h)
          .reshape(_NC, _NR * _L, _CW)[:, :_N])
    cnew = _tc_coord(ctab.reshape(_N, _CW), pc[0], pc[1], 0.0 * pc[0], 0.0 * pc[1])
    return h, cnew[:, :3]


# speedup vs baseline: 1.0861x; 1.0861x over previous
import functools

import jax
import jax.numpy as jnp
from jax import lax
from jax.experimental import pallas as pl
from jax.experimental.pallas import tpu as pltpu
from jax.experimental.pallas import tpu_sc as plsc

_N = 10000
_E = 320000
_H = 128
_NORM_INV = 0.01
_EPS_LN = 1e-5
_EPS_R = 1e-8

_NC = 2
_NS = 16
_NW = _NC * _NS
_L = 16
_C = 128
_NCH = _E // _C
_BASE_CH = _NCH // _NW
_EXTRA = _NCH - _BASE_CH * _NW
_RPT = 624
_CW = 8


def _mesh():
    return plsc.VectorSubcoreMesh(core_axis_name="c", subcore_axis_name="s")


def _worker_id():
    return lax.axis_index("s") * _NC + lax.axis_index("c")


def _foreach_chunk(do_chunk):
    w = _worker_id()

    def body(k, carry):
        do_chunk(w + k * _NW)
        return carry

    lax.fori_loop(0, _BASE_CH, body, 0)

    @pl.when(w < _EXTRA)
    def _():
        do_chunk(_BASE_CH * _NW + w)


def _geom_body(ct_hbm, row_hbm, col_hbm, geom_o, idx_r, idx_c, ct_v, bg, sem):
    pltpu.sync_copy(ct_hbm, ct_v)
    lanes = jnp.arange(_L, dtype=jnp.int32)

    def do_chunk(ci):
        base = ci * _C
        i1 = pltpu.async_copy(row_hbm.at[pl.ds(base, _C)], idx_r, sem)
        i2 = pltpu.async_copy(col_hbm.at[pl.ds(base, _C)], idx_c, sem)
        i1.wait()
        i2.wait()
        for j in range(_C // _L):
            r16 = idx_r[pl.ds(j * _L, _L)] * _CW
            c16 = idx_c[pl.ds(j * _L, _L)] * _CW
            a0 = plsc.load_gather(ct_v, [r16])
            a1 = plsc.load_gather(ct_v, [r16 + 1])
            a2 = plsc.load_gather(ct_v, [r16 + 2])
            b0 = plsc.load_gather(ct_v, [c16])
            b1 = plsc.load_gather(ct_v, [c16 + 1])
            b2 = plsc.load_gather(ct_v, [c16 + 2])
            d0 = a0 - b0
            d1 = a1 - b1
            d2 = a2 - b2
            rad = d0 * d0 + d1 * d1 + d2 * d2
            eb = (j * _L + lanes) * _CW
            plsc.store_scatter(bg, [eb], d0)
            plsc.store_scatter(bg, [eb + 1], d1)
            plsc.store_scatter(bg, [eb + 2], d2)
            plsc.store_scatter(bg, [eb + 3], rad)
        pltpu.sync_copy(bg, geom_o.at[pl.ds(base * _CW, _C * _CW)])

    _foreach_chunk(do_chunk)


def _sc_geom(ctab, row, col):
    f = functools.partial(
        pl.kernel, mesh=_mesh(),
        out_type=jax.ShapeDtypeStruct((_E * _CW,), jnp.float32),
        compiler_params=pltpu.CompilerParams(needs_layout_passes=False),
        scratch_types=[
            pltpu.VMEM((_C,), jnp.int32),
            pltpu.VMEM((_C,), jnp.int32),
            pltpu.VMEM((_N * _CW,), jnp.float32),
            pltpu.VMEM((_C * _CW,), jnp.float32),
            pltpu.SemaphoreType.DMA,
        ],
    )(_geom_body)
    return f(ctab, row, col)


_NB = 3
_GRP = _BASE_CH // _NB

_EH = _E // 2
_HCH = _EH // _C
_HBASE = _HCH // _NW
_HGRP = _HBASE // _NB
_HEXTRA = _HCH - _HBASE * _NW


def _g2_body(h_hbm, row_hbm, col_hbm, src_o, tgt_o,
             idx_r, idx_c, bs, bt, isem, gsem, ws0, ws1, ws2):
    grp, nbase, extra = _HGRP, _HBASE, _HEXTRA
    w = _worker_id()
    wsems = (ws0, ws1, ws2)

    def drain(j):
        pltpu.make_async_copy(bs.at[j], src_o.at[pl.ds(0, _C)], wsems[j]).wait()
        pltpu.make_async_copy(bt.at[j], tgt_o.at[pl.ds(0, _C)], wsems[j]).wait()

    def body(m, carry):
        ih = []
        for j in range(_NB):
            ci = w + (_NB * m + j) * _NW
            base = ci * _C

            @pl.when(m > 0)
            def _():
                drain(j)

            i1 = pltpu.async_copy(row_hbm.at[pl.ds(base, _C)], idx_r.at[j], isem)
            i2 = pltpu.async_copy(col_hbm.at[pl.ds(base, _C)], idx_c.at[j], isem)
            ih.append((i1, i2, base))
        handles = []
        for j in range(_NB):
            i1, i2, base = ih[j]
            i1.wait()
            i2.wait()
            g1 = pltpu.async_copy(h_hbm.at[idx_r.at[j]], bs.at[j], gsem)
            g2 = pltpu.async_copy(h_hbm.at[idx_c.at[j]], bt.at[j], gsem)
            handles.append((g1, g2, base))
        for j in range(_NB):
            g1, g2, base = handles[j]
            g1.wait()
            g2.wait()
            pltpu.async_copy(bs.at[j], src_o.at[pl.ds(base, _C)], wsems[j])
            pltpu.async_copy(bt.at[j], tgt_o.at[pl.ds(base, _C)], wsems[j])
        return carry

    lax.fori_loop(0, grp, body, 0)
    for j in range(_NB):
        drain(j)

    @pl.when(w < extra)
    def _():
        base = (nbase * _NW + w) * _C
        pltpu.sync_copy(row_hbm.at[pl.ds(base, _C)], idx_r.at[0])
        pltpu.sync_copy(col_hbm.at[pl.ds(base, _C)], idx_c.at[0])
        g1 = pltpu.async_copy(h_hbm.at[idx_r.at[0]], bs.at[0], gsem)
        g2 = pltpu.async_copy(h_hbm.at[idx_c.at[0]], bt.at[0], gsem)
        g1.wait()
        g2.wait()
        pltpu.sync_copy(bs.at[0], src_o.at[pl.ds(base, _C)])
        pltpu.sync_copy(bt.at[0], tgt_o.at[pl.ds(base, _C)])


def _gather2(h, row, col):
    f = functools.partial(
        pl.kernel, mesh=_mesh(),
        out_type=[
            jax.ShapeDtypeStruct((_EH, _H), jnp.float32),
            jax.ShapeDtypeStruct((_EH, _H), jnp.float32),
        ],
        scratch_types=[
            pltpu.VMEM((_NB, _C), jnp.int32),
            pltpu.VMEM((_NB, _C), jnp.int32),
            pltpu.VMEM((_NB, _C, _H), jnp.float32),
            pltpu.VMEM((_NB, _C, _H), jnp.float32),
            pltpu.SemaphoreType.DMA,
            pltpu.SemaphoreType.DMA,
            pltpu.SemaphoreType.DMA,
            pltpu.SemaphoreType.DMA,
            pltpu.SemaphoreType.DMA,
        ],
    )(_g2_body)
    return f(h, row, col)


def _scatter_body(ef_hbm, row_hbm, z_hbm, out_hbm, idx_v, buf, acc,
                  isem, lsem, ss0, ss1, ss2):
    c = lax.axis_index("c")
    s = lax.axis_index("s")
    start = s * _RPT

    @pl.when(s < _NS - 1)
    def _():
        pltpu.sync_copy(z_hbm.at[pl.ds(start, _RPT)],
                        acc.at[pl.ds(start, _RPT)])

    @pl.when(s == _NS - 1)
    def _():
        pltpu.sync_copy(z_hbm.at[pl.ds((_NS - 1) * _RPT, _N - (_NS - 1) * _RPT)],
                        acc.at[pl.ds((_NS - 1) * _RPT, _N - (_NS - 1) * _RPT)])

    plsc.subcore_barrier()

    w = s * _NC + c
    ssems = (ss0, ss1, ss2)

    def drain(j):
        pltpu.make_async_copy(buf.at[j], acc.at[pl.ds(0, _C)], ssems[j]).wait()

    def body(m, carry):
        handles = []
        for j in range(_NB):
            ci = w + (_NB * m + j) * _NW
            base = ci * _C

            @pl.when(m > 0)
            def _():
                drain(j)

            i1 = pltpu.async_copy(row_hbm.at[pl.ds(base, _C)], idx_v.at[j], isem)
            h1 = pltpu.async_copy(ef_hbm.at[pl.ds(base, _C)], buf.at[j], lsem)
            handles.append((i1, h1))
        for j in range(_NB):
            i1, h1 = handles[j]
            i1.wait()
            h1.wait()
            pltpu.async_copy(buf.at[j], acc.at[idx_v.at[j]], ssems[j], add=True)
        return carry

    lax.fori_loop(0, _HGRP, body, 0)
    for j in range(_NB):
        drain(j)

    @pl.when(w < _HEXTRA)
    def _():
        base = (_HBASE * _NW + w) * _C
        pltpu.sync_copy(row_hbm.at[pl.ds(base, _C)], idx_v.at[0])
        pltpu.sync_copy(ef_hbm.at[pl.ds(base, _C)], buf.at[0])
        pltpu.sync_copy(buf.at[0], acc.at[idx_v.at[0]], add=True)

    plsc.subcore_barrier()

    @pl.when(s < _NS - 1)
    def _():
        pltpu.sync_copy(acc.at[pl.ds(start, _RPT)],
                        out_hbm.at[c, pl.ds(start, _RPT)])

    @pl.when(s == _NS - 1)
    def _():
        pltpu.sync_copy(acc.at[pl.ds((_NS - 1) * _RPT, _N - (_NS - 1) * _RPT)],
                        out_hbm.at[c, pl.ds((_NS - 1) * _RPT, _N - (_NS - 1) * _RPT)])


def _scatter_add(ef, row, zeros):
    f = functools.partial(
        pl.kernel, mesh=_mesh(),
        out_type=jax.ShapeDtypeStruct((_NC, _N, _H), jnp.float32),
        scratch_types=[
            pltpu.VMEM((_NB, _C), jnp.int32),
            pltpu.VMEM((_NB, _C, _H), jnp.float32),
            pltpu.VMEM_SHARED((_N, _H), jnp.float32),
            pltpu.SemaphoreType.DMA,
            pltpu.SemaphoreType.DMA,
            pltpu.SemaphoreType.DMA,
            pltpu.SemaphoreType.DMA,
            pltpu.SemaphoreType.DMA,
        ],
    )(_scatter_body)
    return f(ef, row, zeros)


_NR = 640
_RPTC = _NR // _NS


def _cscat_body(t_hbm, g_hbm, row_hbm, z_hbm, out_hbm,
                rb0, rb1, rb2, ix0, ix1, ix2, tb0, tb1, tb2,
                gb0, gb1, gb2, sb0, sb1, sb2, acc, isem, lsem, ss0, ss1, ss2):
    c = lax.axis_index("c")
    s = lax.axis_index("s")
    w = s * _NC + c
    lanes = jnp.arange(_L, dtype=jnp.int32)
    zeros16 = jnp.zeros((_L,), jnp.float32)
    rbs = (rb0, rb1, rb2)
    ixs = (ix0, ix1, ix2)
    tbs = (tb0, tb1, tb2)
    gbs = (gb0, gb1, gb2)
    sbs = (sb0, sb1, sb2)
    ssems = (ss0, ss1, ss2)

    pltpu.sync_copy(z_hbm.at[pl.ds(s * _RPTC, _RPTC)],
                    acc.at[pl.ds(s * _RPTC, _RPTC)])
    for j in range(_NB):
        pltpu.sync_copy(z_hbm.at[pl.ds(0, _C)], sbs[j])
    plsc.subcore_barrier()

    def drain(j):
        pltpu.make_async_copy(sbs[j], acc.at[pl.ds(0, _C)], ssems[j]).wait()

    def unfill(j):
        for g in range(_C // _L):
            e16 = g * _L + lanes
            r16 = rbs[j][pl.ds(g * _L, _L)]
            laneb = (r16 & 15) * _CW
            for cc in range(3):
                plsc.store_scatter(sbs[j], [e16, laneb + cc], zeros16)

    def fill(j):
        for g in range(_C // _L):
            e16 = g * _L + lanes
            r16 = rbs[j][pl.ds(g * _L, _L)]
            ixs[j][pl.ds(g * _L, _L)] = lax.shift_right_logical(r16, 4)
            laneb = (r16 & 15) * _CW
            t16 = tbs[j][pl.ds(g * _L, _L)]
            for cc in range(3):
                dcc = plsc.load_gather(gbs[j], [e16 * _CW + cc])
                plsc.store_scatter(sbs[j], [e16, laneb + cc], dcc * t16)

    def body(m, carry):
        handles = []
        for j in range(_NB):
            ci = w + (_NB * m + j) * _NW
            base = ci * _C

            @pl.when(m > 0)
            def _():
                drain(j)
                unfill(j)

            i1 = pltpu.async_copy(row_hbm.at[pl.ds(base, _C)], rbs[j], isem)
            h1 = pltpu.async_copy(t_hbm.at[pl.ds(base, _C)], tbs[j], lsem)
            h2 = pltpu.async_copy(g_hbm.at[pl.ds(base * _CW, _C * _CW)],
                                  gbs[j], lsem)
            handles.append((i1, h1, h2))
        for j in range(_NB):
            i1, h1, h2 = handles[j]
            i1.wait()
            h1.wait()
            h2.wait()
            fill(j)
            pltpu.async_copy(sbs[j], acc.at[ixs[j]], ssems[j], add=True)
        return carry

    lax.fori_loop(0, _HGRP, body, 0)
    for j in range(_NB):
        drain(j)

    @pl.when(w < _HEXTRA)
    def _():
        base = (_HBASE * _NW + w) * _C
        unfill(0)
        pltpu.sync_copy(row_hbm.at[pl.ds(base, _C)], rbs[0])
        pltpu.sync_copy(t_hbm.at[pl.ds(base, _C)], tbs[0])
        pltpu.sync_copy(g_hbm.at[pl.ds(base * _CW, _C * _CW)], gbs[0])
        fill(0)
        pltpu.sync_copy(sbs[0], acc.at[ixs[0]], add=True)

    plsc.subcore_barrier()
    pltpu.sync_copy(acc.at[pl.ds(s * _RPTC, _RPTC)],
                    out_hbm.at[c, pl.ds(s * _RPTC, _RPTC)])


def _coord_scatter(tprime, geom1d, row, zeros):
    f = functools.partial(
        pl.kernel, mesh=_mesh(),
        out_type=jax.ShapeDtypeStruct((_NC, _NR, _H), jnp.float32),
        compiler_params=pltpu.CompilerParams(needs_layout_passes=False),
        scratch_types=(
            [pltpu.VMEM((_C,), jnp.int32)] * _NB
            + [pltpu.VMEM((_C,), jnp.int32)] * _NB
            + [pltpu.VMEM((_C,), jnp.float32)] * _NB
            + [pltpu.VMEM((_C * _CW,), jnp.float32)] * _NB
            + [pltpu.VMEM((_C, _H), jnp.float32)] * _NB
            + [pltpu.VMEM_SHARED((_NR, _H), jnp.float32)]
            + [pltpu.SemaphoreType.DMA] * 5
        ),
    )(_cscat_body)
    return f(tprime, geom1d, row, zeros)


_BE = 2000
_BN = 1000


def _ln_stats(x):
    o = jnp.ones((_H, 1), jnp.float32)
    m = jnp.dot(x, o, preferred_element_type=jnp.float32) * (1.0 / _H)
    s2 = jnp.dot(x * x, o, preferred_element_type=jnp.float32) * (1.0 / _H)
    return m, s2 - m * m


def _edge_body(src_ref, tgt_ref, geom_ref, ea_ref,
               A_ref, B_ref, b1_ref, ar_ref, ae_ref, g1_ref, bg1_ref,
               W2_ref, b2_ref, aw_ref, ab_ref, out_ref):
    radial = geom_ref[...][:, 3:4]
    x = jnp.dot(src_ref[...], A_ref[...], preferred_element_type=jnp.float32)
    x = x + jnp.dot(tgt_ref[...], B_ref[...], preferred_element_type=jnp.float32)
    x = x + radial * ar_ref[...] + ea_ref[...] * ae_ref[...] + b1_ref[...]
    m, v = _ln_stats(x)
    x = (x - m) * lax.rsqrt(v + _EPS_LN) * g1_ref[...] + bg1_ref[...]
    x = x * jax.nn.sigmoid(x)
    y = jnp.dot(x, W2_ref[...],
                preferred_element_type=jnp.float32) + b2_ref[...]
    y = y * jax.nn.sigmoid(y)
    att = jax.nn.sigmoid(
        jnp.dot(y, aw_ref[...], preferred_element_type=jnp.float32)
        + ab_ref[...])
    out_ref[...] = y * att


def _tc_edge(src, tgt, geom, ea, A, B, b1, ar, ae, g1, bg1, W2, b2, aw, ab):
    im = lambda i: (i, 0)
    full = lambda shape: pl.BlockSpec(shape, lambda i: (0, 0))
    return pl.pallas_call(
        _edge_body,
        grid=(_EH // _BE,),
        in_specs=[
            pl.BlockSpec((_BE, _H), im), pl.BlockSpec((_BE, _H), im),
            pl.BlockSpec((_BE, _CW), im), pl.BlockSpec((_BE, 1), im),
            full((_H, _H)), full((_H, _H)), full((1, _H)), full((1, _H)),
            full((1, _H)), full((1, _H)), full((1, _H)),
            full((_H, _H)), full((1, _H)), full((_H, 1)), full((1, 1)),
        ],
        out_specs=pl.BlockSpec((_BE, _H), im),
        out_shape=jax.ShapeDtypeStruct((_EH, _H), jnp.float32),
    )(src, tgt, geom, ea, A, B, b1, ar, ae, g1, bg1, W2, b2, aw, ab)


def _node_body(h_ref, p0_ref, p1_ref, p2_ref, p3_ref,
               Wh_ref, Wa_ref, b1_ref, g_ref, bg_ref, W2_ref, b2_ref, out_ref):
    h = h_ref[...]
    agg = ((p0_ref[...] + p1_ref[...]) + (p2_ref[...] + p3_ref[...])) * _NORM_INV
    x = jnp.dot(h, Wh_ref[...], preferred_element_type=jnp.float32)
    x = x + jnp.dot(agg, Wa_ref[...], preferred_element_type=jnp.float32) + b1_ref[...]
    m = jnp.mean(x, axis=-1, keepdims=True)
    v = jnp.mean((x - m) ** 2, axis=-1, keepdims=True)
    x = (x - m) * lax.rsqrt(v + _EPS_LN) * g_ref[...] + bg_ref[...]
    x = x * jax.nn.sigmoid(x)
    nu = jnp.dot(x, W2_ref[...], preferred_element_type=jnp.float32) + b2_ref[...]
    out_ref[...] = h + nu


def _tc_node(h, pa, pb, Wh, Wa, b1, g, bg, W2, b2):
    im = lambda i: (i, 0)
    full = lambda shape: pl.BlockSpec(shape, lambda i: (0, 0))
    return pl.pallas_call(
        _node_body,
        grid=(_N // _BN,),
        in_specs=[
            pl.BlockSpec((_BN, _H), im), pl.BlockSpec((_BN, _H), im),
            pl.BlockSpec((_BN, _H), im), pl.BlockSpec((_BN, _H), im),
            pl.BlockSpec((_BN, _H), im),
            full((_H, _H)), full((_H, _H)), full((1, _H)), full((1, _H)),
            full((1, _H)), full((_H, _H)), full((1, _H)),
        ],
        out_specs=pl.BlockSpec((_BN, _H), im),
        out_shape=jax.ShapeDtypeStruct((_N, _H), jnp.float32),
    )(h, pa[0], pa[1], pb[0], pb[1], Wh, Wa, b1, g, bg, W2, b2)


def _eq_body(src_ref, tgt_ref, geom_ref, ea_ref,
             A_ref, B_ref, b1_ref, ar_ref, ae_ref, g1_ref, bg1_ref,
             W2_ref, b2_ref, g2_ref, bg2_ref, w3_ref, out_ref):
    geom = geom_ref[...]
    radial = geom[:, 3:4]
    x = jnp.dot(src_ref[...], A_ref[...], preferred_element_type=jnp.float32)
    x = x + jnp.dot(tgt_ref[...], B_ref[...], preferred_element_type=jnp.float32)
    x = x + radial * ar_ref[...] + ea_ref[...] * ae_ref[...] + b1_ref[...]
    m, v = _ln_stats(x)
    x = (x - m) * lax.rsqrt(v + _EPS_LN) * g1_ref[...] + bg1_ref[...]
    x = x * jax.nn.sigmoid(x)
    y = jnp.dot(x, W2_ref[...],
                preferred_element_type=jnp.float32) + b2_ref[...]
    m, v = _ln_stats(y)
    y = (y - m) * lax.rsqrt(v + _EPS_LN) * g2_ref[...] + bg2_ref[...]
    y = y * jax.nn.sigmoid(y)
    t = jnp.dot(y, w3_ref[...], preferred_element_type=jnp.float32)
    out_ref[...] = t / (jnp.sqrt(radial + _EPS_R) + 1.0)


def _tc_eq(src, tgt, geom, ea, A, B, b1, ar, ae, g1, bg1, W2, b2, g2, bg2, w3):
    im = lambda i: (i, 0)
    full = lambda shape: pl.BlockSpec(shape, lambda i: (0, 0))
    return pl.pallas_call(
        _eq_body,
        grid=(_EH // _BE,),
        in_specs=[
            pl.BlockSpec((_BE, _H), im), pl.BlockSpec((_BE, _H), im),
            pl.BlockSpec((_BE, _CW), im), pl.BlockSpec((_BE, 1), im),
            full((_H, _H)), full((_H, _H)), full((1, _H)), full((1, _H)),
            full((1, _H)), full((1, _H)), full((1, _H)),
            full((_H, _H)), full((1, _H)), full((1, _H)), full((1, _H)),
            full((_H, 1)),
        ],
        out_specs=pl.BlockSpec((_BE, 1), im),
        out_shape=jax.ShapeDtypeStruct((_EH, 1), jnp.float32),
    )(src, tgt, geom, ea, A, B, b1, ar, ae, g1, bg1, W2, b2, g2, bg2, w3)


def _coord_body(cp_ref, p0_ref, p1_ref, p2_ref, p3_ref, out_ref):
    out_ref[...] = cp_ref[...] + (
        (p0_ref[...] + p1_ref[...]) + (p2_ref[...] + p3_ref[...])) * _NORM_INV


def _tc_coord(cpad, p0, p1, p2, p3):
    im = lambda i: (i, 0)
    return pl.pallas_call(
        _coord_body,
        grid=(_N // _BN,),
        in_specs=[pl.BlockSpec((_BN, _CW), im)] * 5,
        out_specs=pl.BlockSpec((_BN, _CW), im),
        out_shape=jax.ShapeDtypeStruct((_N, _CW), jnp.float32),
    )(cpad, p0, p1, p2, p3)


def _edge_params(p, w1_key='e_w1', b1_key='e_b1', g_key='e_ln_g', bg_key='e_ln_b',
                 w2_key='e_w2', b2_key='e_b2'):
    W1 = p[w1_key]
    return dict(
        A=W1[:_H], B=W1[_H:2 * _H],
        ar=W1[2 * _H:2 * _H + 1], ae=W1[2 * _H + 1:2 * _H + 2],
        b1=p[b1_key].reshape(1, _H), g1=p[g_key].reshape(1, _H),
        bg1=p[bg_key].reshape(1, _H),
        W2=p[w2_key], b2=p[b2_key].reshape(1, _H),
    )


def kernel(h, coord, edge_attr, params, edge_index):
    row = edge_index[0]
    col = edge_index[1]
    ctab = jnp.pad(coord, ((0, 0), (0, _CW - 3))).reshape(-1)
    zeros_h = jnp.zeros((_N, _H), jnp.float32)

    geom1d = _sc_geom(ctab, row, col)
    halves = []
    for k in range(2):
        sl = slice(k * _EH, (k + 1) * _EH)
        halves.append(dict(
            row=row[sl], col=col[sl], ea=edge_attr[sl],
            geom1d=geom1d[k * _EH * _CW:(k + 1) * _EH * _CW]))

    for i in range(2):
        p = params['gcl%d' % i]
        ep = _edge_params(p)
        gs = [_gather2(h, hv['row'], hv['col']) for hv in halves]
        efs = [_tc_edge(gs[k][0], gs[k][1],
                        halves[k]['geom1d'].reshape(_EH, _CW), halves[k]['ea'],
                        ep['A'], ep['B'], ep['b1'],
                        ep['ar'], ep['ae'], ep['g1'], ep['bg1'],
                        ep['W2'], ep['b2'],
                        p['att_w'], p['att_b'].reshape(1, 1))
               for k in range(2)]
        parts = [_scatter_add(efs[k], halves[k]['row'], zeros_h)
                 for k in range(2)]
        h = _tc_node(h, parts[0], parts[1],
                     p['n_w1'][:_H], p['n_w1'][_H:], p['n_b1'].reshape(1, _H),
                     p['n_ln_g'].reshape(1, _H), p['n_ln_b'].reshape(1, _H),
                     p['n_w2'], p['n_b2'].reshape(1, _H))

    eq = params['eq']
    eqp = _edge_params(eq, w1_key='w1', b1_key='b1', g_key='ln1_g', bg_key='ln1_b',
                       w2_key='w2', b2_key='b2')
    gs = [_gather2(h, hv['row'], hv['col']) for hv in halves]
    tps = [_tc_eq(gs[k][0], gs[k][1],
                  halves[k]['geom1d'].reshape(_EH, _CW), halves[k]['ea'],
                  eqp['A'], eqp['B'], eqp['b1'],
                  eqp['ar'], eqp['ae'], eqp['g1'], eqp['bg1'],
                  eqp['W2'], eqp['b2'],
                  eq['ln2_g'].reshape(1, _H), eq['ln2_b'].reshape(1, _H),
                  eq['w3'])
           for k in range(2)]
    pcs = [_coord_scatter(tps[k].reshape(_EH), halves[k]['geom1d'],
                          halves[k]['row'], zeros_h)
           .reshape(_NC, _NR * _L, _CW)[:, :_N]
           for k in range(2)]
    cnew = _tc_coord(ctab.reshape(_N, _CW), pcs[0][0], pcs[0][1],
                     pcs[1][0], pcs[1][1])
    return h, cnew[:, :3]


# speedup vs baseline: 1.1227x; 1.0338x over previous
import functools

import jax
import jax.numpy as jnp
from jax import lax
from jax.experimental import pallas as pl
from jax.experimental.pallas import tpu as pltpu
from jax.experimental.pallas import tpu_sc as plsc

_N = 10000
_E = 320000
_H = 128
_NORM_INV = 0.01
_EPS_LN = 1e-5
_EPS_R = 1e-8

_NC = 2
_NS = 16
_NW = _NC * _NS
_L = 16
_C = 128
_NCH = _E // _C
_BASE_CH = _NCH // _NW
_EXTRA = _NCH - _BASE_CH * _NW
_RPT = 624
_CW = 8


def _mesh():
    return plsc.VectorSubcoreMesh(core_axis_name="c", subcore_axis_name="s")


def _worker_id():
    return lax.axis_index("s") * _NC + lax.axis_index("c")


def _foreach_chunk(do_chunk):
    w = _worker_id()

    def body(k, carry):
        do_chunk(w + k * _NW)
        return carry

    lax.fori_loop(0, _BASE_CH, body, 0)

    @pl.when(w < _EXTRA)
    def _():
        do_chunk(_BASE_CH * _NW + w)


def _geom_body(ct_hbm, row_hbm, col_hbm, geom_o, idx_r, idx_c, ct_v, bg, sem):
    pltpu.sync_copy(ct_hbm, ct_v)
    lanes = jnp.arange(_L, dtype=jnp.int32)

    def do_chunk(ci):
        base = ci * _C
        i1 = pltpu.async_copy(row_hbm.at[pl.ds(base, _C)], idx_r, sem)
        i2 = pltpu.async_copy(col_hbm.at[pl.ds(base, _C)], idx_c, sem)
        i1.wait()
        i2.wait()
        for j in range(_C // _L):
            r16 = idx_r[pl.ds(j * _L, _L)] * _CW
            c16 = idx_c[pl.ds(j * _L, _L)] * _CW
            a0 = plsc.load_gather(ct_v, [r16])
            a1 = plsc.load_gather(ct_v, [r16 + 1])
            a2 = plsc.load_gather(ct_v, [r16 + 2])
            b0 = plsc.load_gather(ct_v, [c16])
            b1 = plsc.load_gather(ct_v, [c16 + 1])
            b2 = plsc.load_gather(ct_v, [c16 + 2])
            d0 = a0 - b0
            d1 = a1 - b1
            d2 = a2 - b2
            rad = d0 * d0 + d1 * d1 + d2 * d2
            eb = (j * _L + lanes) * _CW
            plsc.store_scatter(bg, [eb], d0)
            plsc.store_scatter(bg, [eb + 1], d1)
            plsc.store_scatter(bg, [eb + 2], d2)
            plsc.store_scatter(bg, [eb + 3], rad)
        pltpu.sync_copy(bg, geom_o.at[pl.ds(base * _CW, _C * _CW)])

    _foreach_chunk(do_chunk)


def _sc_geom(ctab, row, col):
    f = functools.partial(
        pl.kernel, mesh=_mesh(),
        out_type=jax.ShapeDtypeStruct((_E * _CW,), jnp.float32),
        compiler_params=pltpu.CompilerParams(needs_layout_passes=False),
        scratch_types=[
            pltpu.VMEM((_C,), jnp.int32),
            pltpu.VMEM((_C,), jnp.int32),
            pltpu.VMEM((_N * _CW,), jnp.float32),
            pltpu.VMEM((_C * _CW,), jnp.float32),
            pltpu.SemaphoreType.DMA,
        ],
    )(_geom_body)
    return f(ctab, row, col)


_NB = 3
_GRP = _BASE_CH // _NB

_EH = _E // 2
_HCH = _EH // _C
_HBASE = _HCH // _NW
_HGRP = _HBASE // _NB
_HEXTRA = _HCH - _HBASE * _NW


def _g2_body(h_hbm, row_hbm, col_hbm, src_o, tgt_o,
             idx_r, idx_c, bs, bt, isem, gsem, ws0, ws1, ws2):
    grp, nbase, extra = _HGRP, _HBASE, _HEXTRA
    w = _worker_id()
    wsems = (ws0, ws1, ws2)

    def drain(j):
        pltpu.make_async_copy(bs.at[j], src_o.at[pl.ds(0, _C)], wsems[j]).wait()
        pltpu.make_async_copy(bt.at[j], tgt_o.at[pl.ds(0, _C)], wsems[j]).wait()

    def body(m, carry):
        ih = []
        for j in range(_NB):
            ci = w + (_NB * m + j) * _NW
            base = ci * _C

            @pl.when(m > 0)
            def _():
                drain(j)

            i1 = pltpu.async_copy(row_hbm.at[pl.ds(base, _C)], idx_r.at[j], isem)
            i2 = pltpu.async_copy(col_hbm.at[pl.ds(base, _C)], idx_c.at[j], isem)
            ih.append((i1, i2, base))
        handles = []
        for j in range(_NB):
            i1, i2, base = ih[j]
            i1.wait()
            i2.wait()
            g1 = pltpu.async_copy(h_hbm.at[idx_r.at[j]], bs.at[j], gsem)
            g2 = pltpu.async_copy(h_hbm.at[idx_c.at[j]], bt.at[j], gsem)
            handles.append((g1, g2, base))
        for j in range(_NB):
            g1, g2, base = handles[j]
            g1.wait()
            g2.wait()
            pltpu.async_copy(bs.at[j], src_o.at[pl.ds(base, _C)], wsems[j])
            pltpu.async_copy(bt.at[j], tgt_o.at[pl.ds(base, _C)], wsems[j])
        return carry

    lax.fori_loop(0, grp, body, 0)
    for j in range(_NB):
        drain(j)

    @pl.when(w < extra)
    def _():
        base = (nbase * _NW + w) * _C
        pltpu.sync_copy(row_hbm.at[pl.ds(base, _C)], idx_r.at[0])
        pltpu.sync_copy(col_hbm.at[pl.ds(base, _C)], idx_c.at[0])
        g1 = pltpu.async_copy(h_hbm.at[idx_r.at[0]], bs.at[0], gsem)
        g2 = pltpu.async_copy(h_hbm.at[idx_c.at[0]], bt.at[0], gsem)
        g1.wait()
        g2.wait()
        pltpu.sync_copy(bs.at[0], src_o.at[pl.ds(base, _C)])
        pltpu.sync_copy(bt.at[0], tgt_o.at[pl.ds(base, _C)])


def _gather2(h, row, col):
    f = functools.partial(
        pl.kernel, mesh=_mesh(),
        out_type=[
            jax.ShapeDtypeStruct((_EH, _H), jnp.float32),
            jax.ShapeDtypeStruct((_EH, _H), jnp.float32),
        ],
        scratch_types=[
            pltpu.VMEM((_NB, _C), jnp.int32),
            pltpu.VMEM((_NB, _C), jnp.int32),
            pltpu.VMEM((_NB, _C, _H), jnp.float32),
            pltpu.VMEM((_NB, _C, _H), jnp.float32),
            pltpu.SemaphoreType.DMA,
            pltpu.SemaphoreType.DMA,
            pltpu.SemaphoreType.DMA,
            pltpu.SemaphoreType.DMA,
            pltpu.SemaphoreType.DMA,
        ],
    )(_g2_body)
    return f(h, row, col)


def _scatter_body(ef_hbm, row_hbm, z_hbm, out_hbm, idx_v, buf, acc,
                  isem, lsem, ss0, ss1, ss2):
    c = lax.axis_index("c")
    s = lax.axis_index("s")
    start = s * _RPT

    @pl.when(s < _NS - 1)
    def _():
        pltpu.sync_copy(z_hbm.at[pl.ds(start, _RPT)],
                        acc.at[pl.ds(start, _RPT)])

    @pl.when(s == _NS - 1)
    def _():
        pltpu.sync_copy(z_hbm.at[pl.ds((_NS - 1) * _RPT, _N - (_NS - 1) * _RPT)],
                        acc.at[pl.ds((_NS - 1) * _RPT, _N - (_NS - 1) * _RPT)])

    plsc.subcore_barrier()

    w = s * _NC + c
    ssems = (ss0, ss1, ss2)

    def drain(j):
        pltpu.make_async_copy(buf.at[j], acc.at[pl.ds(0, _C)], ssems[j]).wait()

    def body(m, carry):
        handles = []
        for j in range(_NB):
            ci = w + (_NB * m + j) * _NW
            base = ci * _C

            @pl.when(m > 0)
            def _():
                drain(j)

            i1 = pltpu.async_copy(row_hbm.at[pl.ds(base, _C)], idx_v.at[j], isem)
            h1 = pltpu.async_copy(ef_hbm.at[pl.ds(base, _C)], buf.at[j], lsem)
            handles.append((i1, h1))
        for j in range(_NB):
            i1, h1 = handles[j]
            i1.wait()
            h1.wait()
            pltpu.async_copy(buf.at[j], acc.at[idx_v.at[j]], ssems[j], add=True)
        return carry

    lax.fori_loop(0, _HGRP, body, 0)
    for j in range(_NB):
        drain(j)

    @pl.when(w < _HEXTRA)
    def _():
        base = (_HBASE * _NW + w) * _C
        pltpu.sync_copy(row_hbm.at[pl.ds(base, _C)], idx_v.at[0])
        pltpu.sync_copy(ef_hbm.at[pl.ds(base, _C)], buf.at[0])
        pltpu.sync_copy(buf.at[0], acc.at[idx_v.at[0]], add=True)

    plsc.subcore_barrier()

    @pl.when(s < _NS - 1)
    def _():
        pltpu.sync_copy(acc.at[pl.ds(start, _RPT)],
                        out_hbm.at[c, pl.ds(start, _RPT)])

    @pl.when(s == _NS - 1)
    def _():
        pltpu.sync_copy(acc.at[pl.ds((_NS - 1) * _RPT, _N - (_NS - 1) * _RPT)],
                        out_hbm.at[c, pl.ds((_NS - 1) * _RPT, _N - (_NS - 1) * _RPT)])


def _scatter_add(ef, row, zeros):
    f = functools.partial(
        pl.kernel, mesh=_mesh(),
        out_type=jax.ShapeDtypeStruct((_NC, _N, _H), jnp.float32),
        scratch_types=[
            pltpu.VMEM((_NB, _C), jnp.int32),
            pltpu.VMEM((_NB, _C, _H), jnp.float32),
            pltpu.VMEM_SHARED((_N, _H), jnp.float32),
            pltpu.SemaphoreType.DMA,
            pltpu.SemaphoreType.DMA,
            pltpu.SemaphoreType.DMA,
            pltpu.SemaphoreType.DMA,
            pltpu.SemaphoreType.DMA,
        ],
    )(_scatter_body)
    return f(ef, row, zeros)


_NR = 640
_RPTC = _NR // _NS


def _cscat_body(t_hbm, g_hbm, row_hbm, z_hbm, out_hbm,
                rb0, rb1, rb2, ix0, ix1, ix2, tb0, tb1, tb2,
                gb0, gb1, gb2, sb0, sb1, sb2, acc, isem, lsem, ss0, ss1, ss2):
    c = lax.axis_index("c")
    s = lax.axis_index("s")
    w = s * _NC + c
    lanes = jnp.arange(_L, dtype=jnp.int32)
    zeros16 = jnp.zeros((_L,), jnp.float32)
    rbs = (rb0, rb1, rb2)
    ixs = (ix0, ix1, ix2)
    tbs = (tb0, tb1, tb2)
    gbs = (gb0, gb1, gb2)
    sbs = (sb0, sb1, sb2)
    ssems = (ss0, ss1, ss2)

    pltpu.sync_copy(z_hbm.at[pl.ds(s * _RPTC, _RPTC)],
                    acc.at[pl.ds(s * _RPTC, _RPTC)])
    for j in range(_NB):
        pltpu.sync_copy(z_hbm.at[pl.ds(0, _C)], sbs[j])
    plsc.subcore_barrier()

    def drain(j):
        pltpu.make_async_copy(sbs[j], acc.at[pl.ds(0, _C)], ssems[j]).wait()

    def unfill(j):
        for g in range(_C // _L):
            e16 = g * _L + lanes
            r16 = rbs[j][pl.ds(g * _L, _L)]
            laneb = (r16 & 15) * _CW
            for cc in range(3):
                plsc.store_scatter(sbs[j], [e16, laneb + cc], zeros16)

    def fill(j):
        for g in range(_C // _L):
            e16 = g * _L + lanes
            r16 = rbs[j][pl.ds(g * _L, _L)]
            ixs[j][pl.ds(g * _L, _L)] = lax.shift_right_logical(r16, 4)
            laneb = (r16 & 15) * _CW
            t16 = tbs[j][pl.ds(g * _L, _L)]
            for cc in range(3):
                dcc = plsc.load_gather(gbs[j], [e16 * _CW + cc])
                plsc.store_scatter(sbs[j], [e16, laneb + cc], dcc * t16)

    def body(m, carry):
        handles = []
        for j in range(_NB):
            ci = w + (_NB * m + j) * _NW
            base = ci * _C

            @pl.when(m > 0)
            def _():
                drain(j)
                unfill(j)

            i1 = pltpu.async_copy(row_hbm.at[pl.ds(base, _C)], rbs[j], isem)
            h1 = pltpu.async_copy(t_hbm.at[pl.ds(base, _C)], tbs[j], lsem)
            h2 = pltpu.async_copy(g_hbm.at[pl.ds(base * _CW, _C * _CW)],
                                  gbs[j], lsem)
            handles.append((i1, h1, h2))
        for j in range(_NB):
            i1, h1, h2 = handles[j]
            i1.wait()
            h1.wait()
            h2.wait()
            fill(j)
            pltpu.async_copy(sbs[j], acc.at[ixs[j]], ssems[j], add=True)
        return carry

    lax.fori_loop(0, _HGRP, body, 0)
    for j in range(_NB):
        drain(j)

    @pl.when(w < _HEXTRA)
    def _():
        base = (_HBASE * _NW + w) * _C
        unfill(0)
        pltpu.sync_copy(row_hbm.at[pl.ds(base, _C)], rbs[0])
        pltpu.sync_copy(t_hbm.at[pl.ds(base, _C)], tbs[0])
        pltpu.sync_copy(g_hbm.at[pl.ds(base * _CW, _C * _CW)], gbs[0])
        fill(0)
        pltpu.sync_copy(sbs[0], acc.at[ixs[0]], add=True)

    plsc.subcore_barrier()
    pltpu.sync_copy(acc.at[pl.ds(s * _RPTC, _RPTC)],
                    out_hbm.at[c, pl.ds(s * _RPTC, _RPTC)])


def _coord_scatter(tprime, geom1d, row, zeros):
    f = functools.partial(
        pl.kernel, mesh=_mesh(),
        out_type=jax.ShapeDtypeStruct((_NC, _NR, _H), jnp.float32),
        compiler_params=pltpu.CompilerParams(needs_layout_passes=False),
        scratch_types=(
            [pltpu.VMEM((_C,), jnp.int32)] * _NB
            + [pltpu.VMEM((_C,), jnp.int32)] * _NB
            + [pltpu.VMEM((_C,), jnp.float32)] * _NB
            + [pltpu.VMEM((_C * _CW,), jnp.float32)] * _NB
            + [pltpu.VMEM((_C, _H), jnp.float32)] * _NB
            + [pltpu.VMEM_SHARED((_NR, _H), jnp.float32)]
            + [pltpu.SemaphoreType.DMA] * 5
        ),
    )(_cscat_body)
    return f(tprime, geom1d, row, zeros)


_BE = 4000
_BN = 1000


def _ln_stats(x):
    o = jnp.ones((_H, 1), jnp.float32)
    m = jnp.dot(x, o, preferred_element_type=jnp.float32) * (1.0 / _H)
    s2 = jnp.dot(x * x, o, preferred_element_type=jnp.float32) * (1.0 / _H)
    return m, s2 - m * m


def _edge_body(src_ref, tgt_ref, geom_ref, ea_ref,
               A_ref, B_ref, b1_ref, ar_ref, ae_ref, g1_ref, bg1_ref,
               W2_ref, b2_ref, aw_ref, ab_ref, out_ref):
    radial = geom_ref[...][:, 3:4]
    x = jnp.dot(src_ref[...], A_ref[...], preferred_element_type=jnp.float32)
    x = x + jnp.dot(tgt_ref[...], B_ref[...], preferred_element_type=jnp.float32)
    x = x + radial * ar_ref[...] + ea_ref[...] * ae_ref[...] + b1_ref[...]
    m, v = _ln_stats(x)
    x = (x - m) * lax.rsqrt(v + _EPS_LN) * g1_ref[...] + bg1_ref[...]
    x = x * jax.nn.sigmoid(x)
    y = jnp.dot(x, W2_ref[...],
                preferred_element_type=jnp.float32) + b2_ref[...]
    y = y * jax.nn.sigmoid(y)
    att = jax.nn.sigmoid(
        jnp.dot(y, aw_ref[...], preferred_element_type=jnp.float32)
        + ab_ref[...])
    out_ref[...] = y * att


def _tc_edge(src, tgt, geom, ea, A, B, b1, ar, ae, g1, bg1, W2, b2, aw, ab):
    im = lambda i: (i, 0)
    full = lambda shape: pl.BlockSpec(shape, lambda i: (0, 0))
    return pl.pallas_call(
        _edge_body,
        grid=(_EH // _BE,),
        in_specs=[
            pl.BlockSpec((_BE, _H), im), pl.BlockSpec((_BE, _H), im),
            pl.BlockSpec((_BE, _CW), im), pl.BlockSpec((_BE, 1), im),
            full((_H, _H)), full((_H, _H)), full((1, _H)), full((1, _H)),
            full((1, _H)), full((1, _H)), full((1, _H)),
            full((_H, _H)), full((1, _H)), full((_H, 1)), full((1, 1)),
        ],
        out_specs=pl.BlockSpec((_BE, _H), im),
        out_shape=jax.ShapeDtypeStruct((_EH, _H), jnp.float32),
    )(src, tgt, geom, ea, A, B, b1, ar, ae, g1, bg1, W2, b2, aw, ab)


def _node_body(h_ref, p0_ref, p1_ref, p2_ref, p3_ref,
               Wh_ref, Wa_ref, b1_ref, g_ref, bg_ref, W2_ref, b2_ref, out_ref):
    h = h_ref[...]
    agg = ((p0_ref[...] + p1_ref[...]) + (p2_ref[...] + p3_ref[...])) * _NORM_INV
    x = jnp.dot(h, Wh_ref[...], preferred_element_type=jnp.float32)
    x = x + jnp.dot(agg, Wa_ref[...], preferred_element_type=jnp.float32) + b1_ref[...]
    m = jnp.mean(x, axis=-1, keepdims=True)
    v = jnp.mean((x - m) ** 2, axis=-1, keepdims=True)
    x = (x - m) * lax.rsqrt(v + _EPS_LN) * g_ref[...] + bg_ref[...]
    x = x * jax.nn.sigmoid(x)
    nu = jnp.dot(x, W2_ref[...], preferred_element_type=jnp.float32) + b2_ref[...]
    out_ref[...] = h + nu


def _tc_node(h, pa, pb, Wh, Wa, b1, g, bg, W2, b2):
    im = lambda i: (i, 0)
    full = lambda shape: pl.BlockSpec(shape, lambda i: (0, 0))
    return pl.pallas_call(
        _node_body,
        grid=(_N // _BN,),
        in_specs=[
            pl.BlockSpec((_BN, _H), im), pl.BlockSpec((_BN, _H), im),
            pl.BlockSpec((_BN, _H), im), pl.BlockSpec((_BN, _H), im),
            pl.BlockSpec((_BN, _H), im),
            full((_H, _H)), full((_H, _H)), full((1, _H)), full((1, _H)),
            full((1, _H)), full((_H, _H)), full((1, _H)),
        ],
        out_specs=pl.BlockSpec((_BN, _H), im),
        out_shape=jax.ShapeDtypeStruct((_N, _H), jnp.float32),
    )(h, pa[0], pa[1], pb[0], pb[1], Wh, Wa, b1, g, bg, W2, b2)


def _eq_body(src_ref, tgt_ref, geom_ref, ea_ref,
             A_ref, B_ref, b1_ref, ar_ref, ae_ref, g1_ref, bg1_ref,
             W2_ref, b2_ref, g2_ref, bg2_ref, w3_ref, out_ref):
    geom = geom_ref[...]
    radial = geom[:, 3:4]
    x = jnp.dot(src_ref[...], A_ref[...], preferred_element_type=jnp.float32)
    x = x + jnp.dot(tgt_ref[...], B_ref[...], preferred_element_type=jnp.float32)
    x = x + radial * ar_ref[...] + ea_ref[...] * ae_ref[...] + b1_ref[...]
    m, v = _ln_stats(x)
    x = (x - m) * lax.rsqrt(v + _EPS_LN) * g1_ref[...] + bg1_ref[...]
    x = x * jax.nn.sigmoid(x)
    y = jnp.dot(x, W2_ref[...],
                preferred_element_type=jnp.float32) + b2_ref[...]
    m, v = _ln_stats(y)
    y = (y - m) * lax.rsqrt(v + _EPS_LN) * g2_ref[...] + bg2_ref[...]
    y = y * jax.nn.sigmoid(y)
    t = jnp.dot(y, w3_ref[...], preferred_element_type=jnp.float32)
    out_ref[...] = t / (jnp.sqrt(radial + _EPS_R) + 1.0)


def _tc_eq(src, tgt, geom, ea, A, B, b1, ar, ae, g1, bg1, W2, b2, g2, bg2, w3):
    im = lambda i: (i, 0)
    full = lambda shape: pl.BlockSpec(shape, lambda i: (0, 0))
    return pl.pallas_call(
        _eq_body,
        grid=(_EH // _BE,),
        in_specs=[
            pl.BlockSpec((_BE, _H), im), pl.BlockSpec((_BE, _H), im),
            pl.BlockSpec((_BE, _CW), im), pl.BlockSpec((_BE, 1), im),
            full((_H, _H)), full((_H, _H)), full((1, _H)), full((1, _H)),
            full((1, _H)), full((1, _H)), full((1, _H)),
            full((_H, _H)), full((1, _H)), full((1, _H)), full((1, _H)),
            full((_H, 1)),
        ],
        out_specs=pl.BlockSpec((_BE, 1), im),
        out_shape=jax.ShapeDtypeStruct((_EH, 1), jnp.float32),
    )(src, tgt, geom, ea, A, B, b1, ar, ae, g1, bg1, W2, b2, g2, bg2, w3)


def _coord_body(cp_ref, p0_ref, p1_ref, p2_ref, p3_ref, out_ref):
    out_ref[...] = cp_ref[...] + (
        (p0_ref[...] + p1_ref[...]) + (p2_ref[...] + p3_ref[...])) * _NORM_INV


def _tc_coord(cpad, p0, p1, p2, p3):
    im = lambda i: (i, 0)
    return pl.pallas_call(
        _coord_body,
        grid=(_N // _BN,),
        in_specs=[pl.BlockSpec((_BN, _CW), im)] * 5,
        out_specs=pl.BlockSpec((_BN, _CW), im),
        out_shape=jax.ShapeDtypeStruct((_N, _CW), jnp.float32),
    )(cpad, p0, p1, p2, p3)


def _edge_params(p, w1_key='e_w1', b1_key='e_b1', g_key='e_ln_g', bg_key='e_ln_b',
                 w2_key='e_w2', b2_key='e_b2'):
    W1 = p[w1_key]
    return dict(
        A=W1[:_H], B=W1[_H:2 * _H],
        ar=W1[2 * _H:2 * _H + 1], ae=W1[2 * _H + 1:2 * _H + 2],
        b1=p[b1_key].reshape(1, _H), g1=p[g_key].reshape(1, _H),
        bg1=p[bg_key].reshape(1, _H),
        W2=p[w2_key], b2=p[b2_key].reshape(1, _H),
    )


def kernel(h, coord, edge_attr, params, edge_index):
    row = edge_index[0]
    col = edge_index[1]
    ctab = jnp.pad(coord, ((0, 0), (0, _CW - 3))).reshape(-1)
    zeros_h = jnp.zeros((_N, _H), jnp.float32)

    geom1d = _sc_geom(ctab, row, col)
    halves = []
    for k in range(2):
        sl = slice(k * _EH, (k + 1) * _EH)
        halves.append(dict(
            row=row[sl], col=col[sl], ea=edge_attr[sl],
            geom1d=geom1d[k * _EH * _CW:(k + 1) * _EH * _CW]))

    for i in range(2):
        p = params['gcl%d' % i]
        ep = _edge_params(p)
        gs = [_gather2(h, hv['row'], hv['col']) for hv in halves]
        efs = [_tc_edge(gs[k][0], gs[k][1],
                        halves[k]['geom1d'].reshape(_EH, _CW), halves[k]['ea'],
                        ep['A'], ep['B'], ep['b1'],
                        ep['ar'], ep['ae'], ep['g1'], ep['bg1'],
                        ep['W2'], ep['b2'],
                        p['att_w'], p['att_b'].reshape(1, 1))
               for k in range(2)]
        parts = [_scatter_add(efs[k], halves[k]['row'], zeros_h)
                 for k in range(2)]
        h = _tc_node(h, parts[0], parts[1],
                     p['n_w1'][:_H], p['n_w1'][_H:], p['n_b1'].reshape(1, _H),
                     p['n_ln_g'].reshape(1, _H), p['n_ln_b'].reshape(1, _H),
                     p['n_w2'], p['n_b2'].reshape(1, _H))

    eq = params['eq']
    eqp = _edge_params(eq, w1_key='w1', b1_key='b1', g_key='ln1_g', bg_key='ln1_b',
                       w2_key='w2', b2_key='b2')
    gs = [_gather2(h, hv['row'], hv['col']) for hv in halves]
    tps = [_tc_eq(gs[k][0], gs[k][1],
                  halves[k]['geom1d'].reshape(_EH, _CW), halves[k]['ea'],
                  eqp['A'], eqp['B'], eqp['b1'],
                  eqp['ar'], eqp['ae'], eqp['g1'], eqp['bg1'],
                  eqp['W2'], eqp['b2'],
                  eq['ln2_g'].reshape(1, _H), eq['ln2_b'].reshape(1, _H),
                  eq['w3'])
           for k in range(2)]
    pcs = [_coord_scatter(tps[k].reshape(_EH), halves[k]['geom1d'],
                          halves[k]['row'], zeros_h)
           .reshape(_NC, _NR * _L, _CW)[:, :_N]
           for k in range(2)]
    cnew = _tc_coord(ctab.reshape(_N, _CW), pcs[0][0], pcs[0][1],
                     pcs[1][0], pcs[1][1])
    return h, cnew[:, :3]
